# baseline (speedup 1.0000x reference)
import functools

import jax
import jax.numpy as jnp
from jax import lax
from jax.experimental import pallas as pl
from jax.experimental.pallas import tpu as pltpu
from jax.experimental.pallas import tpu_sc as plsc

BS, C, H, W = 64, 3, 512, 512

_DNUMS = lax.GatherDimensionNumbers(
    offset_dims=(), collapsed_slice_dims=(0,), start_index_map=(0,))
NW = 32
IPW = BS // NW
PW = 128
CHUNK = 32768
NCHUNK = (C * H * W) // CHUNK


def _make_sc_kernel():
    mesh = plsc.VectorSubcoreMesh(core_axis_name="c", subcore_axis_name="s")

    @functools.partial(
        pl.kernel, mesh=mesh,
        out_type=jax.ShapeDtypeStruct((BS, C * H * W), jnp.float32),
        scratch_types=[
            pltpu.VMEM((CHUNK,), jnp.float32),
            pltpu.VMEM((PW + W + PW,), jnp.float32),
            pltpu.VMEM((PW + W + PW,), jnp.float32),
            pltpu.VMEM((PW + W + PW,), jnp.float32),
            pltpu.VMEM((W,), jnp.float32),
            pltpu.VMEM((W,), jnp.float32),
            pltpu.VMEM((128,), jnp.float32),
            pltpu.VMEM((128,), jnp.int32),
        ],
    )
    def sc_kernel(x_hbm, scalf_hbm, scali_hbm, out_hbm,
                  chunk_v, row0_v, row1_v, row2_v, orow_v, zrow_v,
                  pf_v, pi_v):
        rows_v = (row0_v, row1_v, row2_v)
        wid = lax.axis_index("s") * 2 + lax.axis_index("c")

        zvec = jnp.zeros((16,), jnp.float32)
        for v in range(W // 16):
            zrow_v[pl.ds(16 * v, 16)] = zvec
        for c in range(C):
            for v in range(PW // 16):
                rows_v[c][pl.ds(16 * v, 16)] = zvec
                rows_v[c][pl.ds(PW + W + 16 * v, 16)] = zvec

        for j in range(IPW):
            b = wid * IPW + j
            pltpu.sync_copy(scalf_hbm.at[b], pf_v)
            pltpu.sync_copy(scali_hbm.at[b], pi_v)
            pf = pf_v[pl.ds(0, 16)]
            pii = pi_v[pl.ds(0, 16)]
            A = pf[0]
            Bc = pf[1]
            cb0 = pf[2]
            cb1 = pf[3]
            dh = pii[0]
            dw = pii[1]

            def p1_chunk(k, acc):
                pltpu.sync_copy(
                    x_hbm.at[b, pl.ds(pl.multiple_of(k * CHUNK, CHUNK), CHUNK)],
                    chunk_v)
                def p1_vec(t, a):
                    return a + chunk_v[pl.ds(16 * t, 16)]
                return lax.fori_loop(0, CHUNK // 16, p1_vec, acc)

            acc = lax.fori_loop(0, NCHUNK, p1_chunk,
                                jnp.zeros((16,), jnp.float32))
            for sh in (1, 2, 4, 8):
                idx = jnp.bitwise_xor(lax.iota(jnp.int32, 16), sh)
                acc = acc + lax.gather(
                    acc, idx[:, None], _DNUMS, (1,),
                    mode=lax.GatherScatterMode.PROMISE_IN_BOUNDS)
            M = acc[0] * (1.0 / (C * H * W))
            Cc = M * cb0 + cb1

            def p2_row(r, carry):
                srow = r + dh
                rvalid = (srow >= 0) & (srow < H)

                @pl.when(rvalid)
                def _interior():
                    for c in range(C):
                        pltpu.sync_copy(
                            x_hbm.at[b, pl.ds(
                                pl.multiple_of(c * H * W + srow * W, W), W)],
                            rows_v[c].at[pl.ds(PW, W)])
                    for c in range(C):
                        for v in range(W // 16):
                            off = PW + dw + 16 * v
                            x0 = row0_v[pl.ds(off, 16)]
                            x1 = row1_v[pl.ds(off, 16)]
                            x2 = row2_v[pl.ds(off, 16)]
                            mcv = (x0 + x1 + x2) * (1.0 / 3.0)
                            xc = rows_v[c][pl.ds(off, 16)]
                            o = A * xc + Bc * mcv + Cc
                            colpos = lax.iota(jnp.int32, 16) + (16 * v + dw)
                            m = (colpos >= 0) & (colpos < W)
                            orow_v[pl.ds(16 * v, 16)] = jnp.where(m, o, 0.0)
                        pltpu.sync_copy(
                            orow_v,
                            out_hbm.at[b, pl.ds(
                                pl.multiple_of(c * H * W + r * W, W), W)])

                @pl.when(jnp.logical_not(rvalid))
                def _fill():
                    for c in range(C):
                        pltpu.sync_copy(
                            zrow_v,
                            out_hbm.at[b, pl.ds(
                                pl.multiple_of(c * H * W + r * W, W), W)])

                return carry

            lax.fori_loop(0, H, p2_row, 0)

    return sc_kernel


@jax.jit
def kernel(x, b_rand, s_rand, c_rand, dh, dw):
    br = b_rand.reshape(BS)
    sr = s_rand.reshape(BS)
    cr = c_rand.reshape(BS)
    cs = cr + 0.5
    scalf = jnp.stack([cs * 2.0 * sr, cs * (1.0 - 2.0 * sr),
                       0.5 - cr, br - 0.5], axis=1)
    scalf = jnp.pad(scalf, ((0, 0), (0, 124))).astype(jnp.float32)
    scali = jnp.stack([dh.reshape(BS), dw.reshape(BS)], axis=1)
    scali = jnp.pad(scali, ((0, 0), (0, 126))).astype(jnp.int32)
    out = _make_sc_kernel()(x.reshape(BS, C * H * W), scalf, scali)
    return out.reshape(BS, C, H, W)


# speedup vs baseline: 25.0999x; 25.0999x over previous
import jax
import jax.numpy as jnp
from jax import lax
from jax.experimental import pallas as pl
from jax.experimental.pallas import tpu as pltpu

BS, C, H, W = 64, 3, 512, 512


PR = 64
SH = H + 2 * PR


IMG_PER_STEP = 2


def _diffaug_kernel(br_ref, sr_ref, cr_ref, dh_ref, dw_ref, x_ref, o_ref,
                    s_ref):
    i = pl.program_id(0)

    @pl.when(i == 0)
    def _zero_pads():
        s_ref[...] = jnp.zeros_like(s_ref)

    for j in range(IMG_PER_STEP):
        b = i * IMG_PER_STEP + j
        br = br_ref[b]
        sr = sr_ref[b]
        cr = cr_ref[b]
        dh = dh_ref[b]
        dw = dw_ref[b]

        xb = x_ref[j]
        mc = (xb[0] + xb[1] + xb[2]) * (1.0 / 3.0)
        M = jnp.mean(mc)

        cs = cr + 0.5
        A = cs * 2.0 * sr
        B = cs * (1.0 - 2.0 * sr)
        Cc = M * (0.5 - cr) + br - 0.5
        t = B * mc + Cc
        o3 = A * xb + t[None, :, :]

        cols = lax.broadcasted_iota(jnp.int32, (H, W), 1)
        cvalid = (cols + dw >= 0) & (cols + dw < W)
        o3 = jnp.where(cvalid[None, :, :], pltpu.roll(o3, -dw, 2), 0.0)

        s_ref[:, PR:PR + H, :] = o3
        start = PR + dh
        rr = lax.rem(start, 8)
        base = pl.multiple_of(start - rr, 8)
        for r in range(8):
            @pl.when(rr == r)
            def _copy(r=r, j=j):
                v = s_ref[:, pl.ds(base, H + 8), :]
                o_ref[j] = v[:, r:r + H, :]


@jax.jit
def kernel(x, b_rand, s_rand, c_rand, dh, dw):
    br = b_rand.reshape(BS).astype(jnp.float32)
    sr = s_rand.reshape(BS).astype(jnp.float32)
    cr = c_rand.reshape(BS).astype(jnp.float32)
    dhi = dh.reshape(BS).astype(jnp.int32)
    dwi = dw.reshape(BS).astype(jnp.int32)

    grid_spec = pltpu.PrefetchScalarGridSpec(
        num_scalar_prefetch=5,
        grid=(BS // IMG_PER_STEP,),
        in_specs=[
            pl.BlockSpec((IMG_PER_STEP, C, H, W), lambda i, *_: (i, 0, 0, 0)),
        ],
        out_specs=pl.BlockSpec((IMG_PER_STEP, C, H, W),
                               lambda i, *_: (i, 0, 0, 0)),
        scratch_shapes=[pltpu.VMEM((C, SH, W), jnp.float32)],
    )
    return pl.pallas_call(
        _diffaug_kernel,
        grid_spec=grid_spec,
        out_shape=jax.ShapeDtypeStruct((BS, C, H, W), jnp.float32),
    )(br, sr, cr, dhi, dwi, x)


# speedup vs baseline: 26.4110x; 1.0522x over previous
import jax
import jax.numpy as jnp
from jax import lax
from jax.experimental import pallas as pl
from jax.experimental.pallas import tpu as pltpu

BS, C, H, W = 64, 3, 512, 512


PR = 64
SH = H + 2 * PR


IMG_PER_STEP = 4


def _diffaug_kernel(br_ref, sr_ref, cr_ref, dh_ref, dw_ref, x_ref, o_ref,
                    s_ref):
    i = pl.program_id(0)

    @pl.when(i == 0)
    def _zero_pads():
        s_ref[...] = jnp.zeros_like(s_ref)

    for j in range(IMG_PER_STEP):
        b = i * IMG_PER_STEP + j
        br = br_ref[b]
        sr = sr_ref[b]
        cr = cr_ref[b]
        dh = dh_ref[b]
        dw = dw_ref[b]

        xb = x_ref[j]
        mc = (xb[0] + xb[1] + xb[2]) * (1.0 / 3.0)
        M = jnp.mean(mc)

        cs = cr + 0.5
        A = cs * 2.0 * sr
        B = cs * (1.0 - 2.0 * sr)
        Cc = M * (0.5 - cr) + br - 0.5
        t = B * mc + Cc
        o3 = A * xb + t[None, :, :]

        cols = lax.broadcasted_iota(jnp.int32, (H, W), 1)
        cvalid = (cols + dw >= 0) & (cols + dw < W)
        o3 = jnp.where(cvalid[None, :, :], pltpu.roll(o3, -dw, 2), 0.0)

        s_ref[:, PR:PR + H, :] = o3
        start = PR + dh
        rr = lax.rem(start, 8)
        base = pl.multiple_of(start - rr, 8)
        for r in range(8):
            @pl.when(rr == r)
            def _copy(r=r, j=j):
                v = s_ref[:, pl.ds(base, H + 8), :]
                o_ref[j] = v[:, r:r + H, :]


@jax.jit
def kernel(x, b_rand, s_rand, c_rand, dh, dw):
    br = b_rand.reshape(BS).astype(jnp.float32)
    sr = s_rand.reshape(BS).astype(jnp.float32)
    cr = c_rand.reshape(BS).astype(jnp.float32)
    dhi = dh.reshape(BS).astype(jnp.int32)
    dwi = dw.reshape(BS).astype(jnp.int32)

    grid_spec = pltpu.PrefetchScalarGridSpec(
        num_scalar_prefetch=5,
        grid=(BS // IMG_PER_STEP,),
        in_specs=[
            pl.BlockSpec((IMG_PER_STEP, C, H, W), lambda i, *_: (i, 0, 0, 0)),
        ],
        out_specs=pl.BlockSpec((IMG_PER_STEP, C, H, W),
                               lambda i, *_: (i, 0, 0, 0)),
        scratch_shapes=[pltpu.VMEM((C, SH, W), jnp.float32)],
    )
    return pl.pallas_call(
        _diffaug_kernel,
        grid_spec=grid_spec,
        out_shape=jax.ShapeDtypeStruct((BS, C, H, W), jnp.float32),
    )(br, sr, cr, dhi, dwi, x)
